# baseline (speedup 1.0000x reference)
import functools

import jax
import jax.numpy as jnp
from jax import lax
from jax.experimental import pallas as pl
from jax.experimental.pallas import tpu as pltpu
from jax.experimental.pallas import tpu_sc as plsc

N = 100000
D = 128
M = 25000

NUM_CORES = 2
NUM_SUBCORES = 16
NW = NUM_CORES * NUM_SUBCORES
B_PER_W = 784
M_PAD = B_PER_W * NW
CHUNK = 112
N_CHUNKS = B_PER_W // CHUNK

_MESH = plsc.VectorSubcoreMesh(core_axis_name="c", subcore_axis_name="s")


@functools.partial(
    pl.kernel,
    out_type=jax.ShapeDtypeStruct((M_PAD, D), jnp.float32),
    mesh=_MESH,
    scratch_types=[
        pltpu.VMEM((B_PER_W,), jnp.int32),
        pltpu.VMEM((B_PER_W, D), jnp.float32),
        pltpu.SemaphoreType.DMA,
    ],
)
def _sc_gather(feat_hbm, idx_hbm, out_hbm, idx_v, rows_v, sem):
    wid = lax.axis_index("s") * NUM_CORES + lax.axis_index("c")
    base = wid * B_PER_W
    pltpu.sync_copy(idx_hbm.at[pl.ds(base, B_PER_W)], idx_v)
    copies = []
    for j in range(N_CHUNKS):
        copies.append(
            pltpu.async_copy(
                feat_hbm.at[idx_v.at[pl.ds(j * CHUNK, CHUNK)]],
                rows_v.at[pl.ds(j * CHUNK, CHUNK)],
                sem,
            )
        )
    for c in copies:
        c.wait()
    pltpu.sync_copy(rows_v, out_hbm.at[pl.ds(base, B_PER_W)])


def _mm_body(g_ref, w_ref, b_ref, o_ref):
    o_ref[...] = (
        lax.dot_general(
            g_ref[...], w_ref[...], (((1,), (1,)), ((), ())),
            preferred_element_type=jnp.float32,
        )
        + b_ref[...]
    )


_TM = 25088


def _tc_matmul(gathered, w2, b2):
    return pl.pallas_call(
        _mm_body,
        grid=(M_PAD // _TM,),
        in_specs=[
            pl.BlockSpec((_TM, D), lambda i: (i, 0)),
            pl.BlockSpec((D, D), lambda i: (0, 0)),
            pl.BlockSpec((1, D), lambda i: (0, 0)),
        ],
        out_specs=pl.BlockSpec((_TM, D), lambda i: (i, 0)),
        out_shape=jax.ShapeDtypeStruct((M, D), jnp.float32),
    )(gathered, w2, b2)


def kernel(feat, gtensor, itensor, idi, W, b):
    del gtensor, itensor
    d_out = W.shape[0]
    d_in = W.shape[-1]
    idx_pad = jnp.concatenate(
        [idi, jnp.arange(M_PAD - M, dtype=jnp.int32)]
    )
    gathered = _sc_gather(feat, idx_pad)
    w2 = W.reshape(d_out, d_in)
    b2 = b.reshape(1, D)
    return _tc_matmul(gathered, w2, b2)


# speedup vs baseline: 1.0621x; 1.0621x over previous
import functools

import jax
import jax.numpy as jnp
from jax import lax
from jax.experimental import pallas as pl
from jax.experimental.pallas import tpu as pltpu
from jax.experimental.pallas import tpu_sc as plsc

N = 100000
D = 128
M = 25000

NUM_CORES = 2
NUM_SUBCORES = 16
NW = NUM_CORES * NUM_SUBCORES
B_PER_W = 784
M_PAD = B_PER_W * NW
CHUNK = 112
N_CHUNKS = B_PER_W // CHUNK

_MESH = plsc.VectorSubcoreMesh(core_axis_name="c", subcore_axis_name="s")


@functools.partial(
    pl.kernel,
    out_type=jax.ShapeDtypeStruct((M_PAD, D), jnp.float32),
    mesh=_MESH,
    scratch_types=[
        pltpu.VMEM((B_PER_W,), jnp.int32),
        pltpu.VMEM((B_PER_W, D), jnp.float32),
        [pltpu.SemaphoreType.DMA] * N_CHUNKS,
        pltpu.SemaphoreType.DMA,
    ],
)
def _sc_gather(feat_hbm, idx_hbm, out_hbm, idx_v, rows_v, gsems, store_sem):
    wid = lax.axis_index("s") * NUM_CORES + lax.axis_index("c")
    base = wid * B_PER_W
    pltpu.sync_copy(idx_hbm.at[pl.ds(base, B_PER_W)], idx_v)
    gathers = []
    for j in range(N_CHUNKS):
        gathers.append(
            pltpu.async_copy(
                feat_hbm.at[idx_v.at[pl.ds(j * CHUNK, CHUNK)]],
                rows_v.at[pl.ds(j * CHUNK, CHUNK)],
                gsems[j],
            )
        )
    stores = []
    for j in range(N_CHUNKS):
        gathers[j].wait()
        stores.append(
            pltpu.async_copy(
                rows_v.at[pl.ds(j * CHUNK, CHUNK)],
                out_hbm.at[pl.ds(base + j * CHUNK, CHUNK)],
                store_sem,
            )
        )
    for s in stores:
        s.wait()


def _mm_body(g_ref, w_ref, b_ref, o_ref):
    o_ref[...] = (
        lax.dot_general(
            g_ref[...], w_ref[...], (((1,), (1,)), ((), ())),
            preferred_element_type=jnp.float32,
        )
        + b_ref[...]
    )


_TM = 12544


def _tc_matmul(gathered, w2, b2):
    return pl.pallas_call(
        _mm_body,
        grid=(M_PAD // _TM,),
        in_specs=[
            pl.BlockSpec((_TM, D), lambda i: (i, 0)),
            pl.BlockSpec((D, D), lambda i: (0, 0)),
            pl.BlockSpec((1, D), lambda i: (0, 0)),
        ],
        out_specs=pl.BlockSpec((_TM, D), lambda i: (i, 0)),
        out_shape=jax.ShapeDtypeStruct((M, D), jnp.float32),
    )(gathered, w2, b2)


def kernel(feat, gtensor, itensor, idi, W, b):
    del gtensor, itensor
    d_out = W.shape[0]
    d_in = W.shape[-1]
    idx_pad = jnp.concatenate(
        [idi, jnp.arange(M_PAD - M, dtype=jnp.int32)]
    )
    gathered = _sc_gather(feat, idx_pad)
    w2 = W.reshape(d_out, d_in)
    b2 = b.reshape(1, D)
    return _tc_matmul(gathered, w2, b2)
